# baseline (speedup 1.0000x reference)
import functools

import jax
import jax.numpy as jnp
from jax import lax
from jax.experimental import pallas as pl
from jax.experimental.pallas import tpu as pltpu
from jax.experimental.pallas import tpu_sc as plsc

B, S = 4096, 50
N = B * S
FD, RD, OUT = 32, 32, 128
NR = 50
KP = 52
GW = KP * FD

NC, NS = 2, 16
NW = NC * NS
BATCH_W = B // NW
NSUPER = 4
BATCH_SUP = BATCH_W // NSUPER
NCHUNK = 4
BATCH_CH = BATCH_SUP // NCHUNK
CH = BATCH_CH * S
GROWS_SUP = BATCH_SUP * KP
NZB = GROWS_SUP // 832
NSCAT = 5
SCAT = CH // NSCAT

NB_BLK = 128
GRID = B // NB_BLK
PAD_BLK = 512


def _sc_bind(table, f_idx, r_idx):
    mesh = plsc.VectorSubcoreMesh(core_axis_name="c", subcore_axis_name="s")

    @functools.partial(
        pl.kernel,
        out_type=jax.ShapeDtypeStruct((B * KP, FD), jnp.float32),
        mesh=mesh,
        scratch_types=[
            pltpu.VMEM((BATCH_CH, 128), jnp.int32),
            pltpu.VMEM((BATCH_CH, 128), jnp.int32),
            pltpu.VMEM((CH,), jnp.int32),
            pltpu.VMEM((CH,), jnp.int32),
            pltpu.VMEM((NSCAT, SCAT), jnp.int32),
            pltpu.VMEM((CH, FD), jnp.float32),
            pltpu.VMEM((832, FD), jnp.float32),
            pltpu.VMEM_SHARED((NS, GROWS_SUP, FD), jnp.float32),
        ],
        compiler_params=pltpu.CompilerParams(use_tc_tiling_on_sc=False),
    )
    def k(tab_hbm, fi_hbm, ri_hbm, g_hbm,
          fpad_v, rpad_v, fidx_v, ridx_v, tgt_v, rows_v, zeros_v, acc_sh):
        cid = lax.axis_index("c")
        sid = lax.axis_index("s")
        wid = sid * NC + cid
        iota16 = lax.broadcasted_iota(jnp.int32, (16,), 0)
        z16 = jnp.zeros((16,), jnp.float32)

        @pl.loop(0, 832)
        def _(i):
            zeros_v[i, pl.ds(0, 16)] = z16
            zeros_v[i, pl.ds(16, 16)] = z16

        acc = acc_sh.at[sid]

        @pl.loop(0, NSUPER)
        def _(h):
            b_sup = wid * BATCH_W + h * BATCH_SUP

            @pl.loop(0, NZB)
            def _(zb):
                pltpu.sync_copy(zeros_v, acc.at[pl.ds(zb * 832, 832)])

            @pl.loop(0, NCHUNK)
            def _(cc):
                b0 = b_sup + cc * BATCH_CH
                pltpu.sync_copy(fi_hbm.at[pl.ds(b0, BATCH_CH)], fpad_v)
                pltpu.sync_copy(ri_hbm.at[pl.ds(b0, BATCH_CH)], rpad_v)

                @pl.loop(0, BATCH_CH)
                def _(bi):
                    for c in (0, 16, 32, 34):
                        fidx_v[pl.ds(bi * S + c, 16)] = fpad_v[bi, pl.ds(c, 16)]
                        ridx_v[pl.ds(bi * S + c, 16)] = rpad_v[bi, pl.ds(c, 16)]

                pltpu.sync_copy(tab_hbm.at[fidx_v], rows_v)

                @pl.loop(0, NSCAT)
                def _(j):
                    for t in range(SCAT // 16):
                        r0 = j * SCAT + t * 16
                        role16 = ridx_v[pl.ds(r0, 16)]
                        bloc = lax.shift_right_logical(
                            (r0 + iota16) * 1311, 16)
                        tgt = bloc * KP + cc * (BATCH_CH * KP) + role16
                        tgt_v[j, pl.ds(t * 16, 16)] = tgt

                @pl.loop(0, NSCAT)
                def _(j):
                    pltpu.sync_copy(rows_v.at[pl.ds(j * SCAT, SCAT)],
                                    acc.at[tgt_v.at[j]], add=True)

            pltpu.sync_copy(acc, g_hbm.at[pl.ds(b_sup * KP, GROWS_SUP)])

    return k(table, f_idx, r_idx)


def _pad_body(f_ref, r_ref, fo_ref, ro_ref):
    z = jnp.zeros((PAD_BLK, 128 - S), jnp.int32)
    fo_ref[...] = jnp.concatenate([f_ref[...], z], axis=1)
    ro_ref[...] = jnp.concatenate([r_ref[...], z], axis=1)


def _pad_idx(filler_list, role_list):
    return pl.pallas_call(
        _pad_body,
        grid=(B // PAD_BLK,),
        in_specs=[
            pl.BlockSpec((PAD_BLK, S), lambda i: (i, 0)),
            pl.BlockSpec((PAD_BLK, S), lambda i: (i, 0)),
        ],
        out_specs=[
            pl.BlockSpec((PAD_BLK, 128), lambda i: (i, 0)),
            pl.BlockSpec((PAD_BLK, 128), lambda i: (i, 0)),
        ],
        out_shape=[jax.ShapeDtypeStruct((B, 128), jnp.int32),
                   jax.ShapeDtypeStruct((B, 128), jnp.int32)],
    )(filler_list, role_list)


def _tc_body(g_ref, remb_ref, w_ref, b_ref, o_ref, m_scr):
    @pl.when(pl.program_id(0) == 0)
    def _():
        remb_rep = jnp.broadcast_to(
            remb_ref[...][:, None, :], (NR, FD, RD)).reshape(NR * FD, RD)
        rr = lax.broadcasted_iota(jnp.int32, (RD, FD * RD), 0)
        cc = lax.broadcasted_iota(jnp.int32, (RD, FD * RD), 1)
        tmat = (rr == cc % RD).astype(jnp.float32)
        raw = jnp.dot(remb_rep, tmat, preferred_element_type=jnp.float32)
        mr = lax.broadcasted_iota(jnp.int32, (NR * FD, FD * RD), 0)
        mc = lax.broadcasted_iota(jnp.int32, (NR * FD, FD * RD), 1)
        e_all = raw * (mr % FD == mc // RD).astype(jnp.float32)
        m_scr[...] = jnp.zeros((GW, OUT), jnp.float32)
        m_scr[pl.ds(0, NR * FD), :] = lax.dot_general(
            e_all, w_ref[...], (((1,), (1,)), ((), ())),
            preferred_element_type=jnp.float32)

    o_ref[...] = jnp.dot(g_ref[...], m_scr[...],
                         preferred_element_type=jnp.float32) + b_ref[...]


def _tc_compute(g2, role_emb, w, b2):
    return pl.pallas_call(
        _tc_body,
        grid=(GRID,),
        in_specs=[
            pl.BlockSpec((NB_BLK, GW), lambda i: (i, 0)),
            pl.BlockSpec((NR, RD), lambda i: (0, 0)),
            pl.BlockSpec((OUT, FD * RD), lambda i: (0, 0)),
            pl.BlockSpec((1, OUT), lambda i: (0, 0)),
        ],
        out_specs=pl.BlockSpec((NB_BLK, OUT), lambda i: (i, 0)),
        out_shape=jax.ShapeDtypeStruct((B, OUT), jnp.float32),
        scratch_shapes=[pltpu.VMEM((GW, OUT), jnp.float32)],
    )(g2, role_emb, w, b2)


@jax.jit
def kernel(filler_list, role_list, filler_emb, role_emb, W, b):
    fpad, rpad = _pad_idx(filler_list, role_list)
    g = _sc_bind(filler_emb, fpad, rpad)
    return _tc_compute(g.reshape(B, GW), role_emb, W, b.reshape(1, -1))


# speedup vs baseline: 1.0043x; 1.0043x over previous
import functools

import jax
import jax.numpy as jnp
from jax import lax
from jax.experimental import pallas as pl
from jax.experimental.pallas import tpu as pltpu
from jax.experimental.pallas import tpu_sc as plsc

B, S = 4096, 50
N = B * S
FD, RD, OUT = 32, 32, 128
NR = 50
KP = 52
GW = KP * FD

NC, NS = 2, 16
NW = NC * NS
BATCH_W = B // NW
NSUPER = 4
BATCH_SUP = BATCH_W // NSUPER
NCHUNK = 4
BATCH_CH = BATCH_SUP // NCHUNK
CH = BATCH_CH * S
GROWS_SUP = BATCH_SUP * KP
NZB = GROWS_SUP // 832
NSCAT = 5
SCAT = CH // NSCAT

NB_BLK = 128
GRID = B // NB_BLK
PAD_BLK = 512


def _sc_densify(fpad, rpad):
    mesh = plsc.VectorSubcoreMesh(core_axis_name="c", subcore_axis_name="s")

    @functools.partial(
        pl.kernel,
        out_type=(jax.ShapeDtypeStruct((N,), jnp.int32),
                  jax.ShapeDtypeStruct((N,), jnp.int32)),
        mesh=mesh,
        scratch_types=[
            pltpu.VMEM((BATCH_CH, 128), jnp.int32),
            pltpu.VMEM((BATCH_CH, 128), jnp.int32),
            pltpu.VMEM((CH,), jnp.int32),
            pltpu.VMEM((CH,), jnp.int32),
        ],
        compiler_params=pltpu.CompilerParams(use_tc_tiling_on_sc=True),
    )
    def k(fi_hbm, ri_hbm, fo_hbm, ro_hbm, fpad_v, rpad_v, fdense_v, rdense_v):
        cid = lax.axis_index("c")
        sid = lax.axis_index("s")
        wid = sid * NC + cid

        @pl.loop(0, BATCH_W // BATCH_CH)
        def _(cc):
            b0 = wid * BATCH_W + cc * BATCH_CH
            pltpu.sync_copy(fi_hbm.at[pl.ds(b0, BATCH_CH)], fpad_v)
            pltpu.sync_copy(ri_hbm.at[pl.ds(b0, BATCH_CH)], rpad_v)

            @pl.loop(0, BATCH_CH)
            def _(bi):
                for c in (0, 16, 32, 34):
                    fdense_v[pl.ds(bi * S + c, 16)] = fpad_v[bi, pl.ds(c, 16)]
                    rdense_v[pl.ds(bi * S + c, 16)] = rpad_v[bi, pl.ds(c, 16)]

            pltpu.sync_copy(fdense_v, fo_hbm.at[pl.ds(b0 * S, CH)])
            pltpu.sync_copy(rdense_v, ro_hbm.at[pl.ds(b0 * S, CH)])

    return k(fpad, rpad)

def _sc_bind(table, f_idx, r_idx):
    mesh = plsc.VectorSubcoreMesh(core_axis_name="c", subcore_axis_name="s")

    @functools.partial(
        pl.kernel,
        out_type=jax.ShapeDtypeStruct((B * KP, FD), jnp.float32),
        mesh=mesh,
        scratch_types=[
            pltpu.VMEM((CH,), jnp.int32),
            pltpu.VMEM((CH,), jnp.int32),
            pltpu.VMEM((NSCAT, SCAT), jnp.int32),
            pltpu.VMEM((CH, FD), jnp.float32),
            pltpu.VMEM((832, FD), jnp.float32),
            pltpu.VMEM_SHARED((NS, GROWS_SUP, FD), jnp.float32),
        ],
        compiler_params=pltpu.CompilerParams(use_tc_tiling_on_sc=False),
    )
    def k(tab_hbm, fi_hbm, ri_hbm, g_hbm,
          fidx_v, ridx_v, tgt_v, rows_v, zeros_v, acc_sh):
        cid = lax.axis_index("c")
        sid = lax.axis_index("s")
        wid = sid * NC + cid
        iota16 = lax.broadcasted_iota(jnp.int32, (16,), 0)
        z16 = jnp.zeros((16,), jnp.float32)

        @pl.loop(0, 832)
        def _(i):
            zeros_v[i, pl.ds(0, 16)] = z16
            zeros_v[i, pl.ds(16, 16)] = z16

        acc = acc_sh.at[sid]

        @pl.loop(0, NSUPER)
        def _(h):
            b_sup = wid * BATCH_W + h * BATCH_SUP

            @pl.loop(0, NZB)
            def _(zb):
                pltpu.sync_copy(zeros_v, acc.at[pl.ds(zb * 832, 832)])

            @pl.loop(0, NCHUNK)
            def _(cc):
                goff = (b_sup + cc * BATCH_CH) * S
                pltpu.sync_copy(fi_hbm.at[pl.ds(goff, CH)], fidx_v)
                pltpu.sync_copy(ri_hbm.at[pl.ds(goff, CH)], ridx_v)
                pltpu.sync_copy(tab_hbm.at[fidx_v], rows_v)

                @pl.loop(0, NSCAT)
                def _(j):
                    for t in range(SCAT // 16):
                        r0 = j * SCAT + t * 16
                        role16 = ridx_v[pl.ds(r0, 16)]
                        bloc = lax.shift_right_logical(
                            (r0 + iota16) * 1311, 16)
                        tgt = bloc * KP + cc * (BATCH_CH * KP) + role16
                        tgt_v[j, pl.ds(t * 16, 16)] = tgt

                @pl.loop(0, NSCAT)
                def _(j):
                    pltpu.sync_copy(rows_v.at[pl.ds(j * SCAT, SCAT)],
                                    acc.at[tgt_v.at[j]], add=True)

            pltpu.sync_copy(acc, g_hbm.at[pl.ds(b_sup * KP, GROWS_SUP)])

    return k(table, f_idx, r_idx)


def _pad_body(f_ref, r_ref, fo_ref, ro_ref):
    z = jnp.zeros((PAD_BLK, 128 - S), jnp.int32)
    fo_ref[...] = jnp.concatenate([f_ref[...], z], axis=1)
    ro_ref[...] = jnp.concatenate([r_ref[...], z], axis=1)


def _pad_idx(filler_list, role_list):
    return pl.pallas_call(
        _pad_body,
        grid=(B // PAD_BLK,),
        in_specs=[
            pl.BlockSpec((PAD_BLK, S), lambda i: (i, 0)),
            pl.BlockSpec((PAD_BLK, S), lambda i: (i, 0)),
        ],
        out_specs=[
            pl.BlockSpec((PAD_BLK, 128), lambda i: (i, 0)),
            pl.BlockSpec((PAD_BLK, 128), lambda i: (i, 0)),
        ],
        out_shape=[jax.ShapeDtypeStruct((B, 128), jnp.int32),
                   jax.ShapeDtypeStruct((B, 128), jnp.int32)],
    )(filler_list, role_list)


def _tc_body(g_ref, remb_ref, w_ref, b_ref, o_ref, m_scr):
    @pl.when(pl.program_id(0) == 0)
    def _():
        remb_rep = jnp.broadcast_to(
            remb_ref[...][:, None, :], (NR, FD, RD)).reshape(NR * FD, RD)
        rr = lax.broadcasted_iota(jnp.int32, (RD, FD * RD), 0)
        cc = lax.broadcasted_iota(jnp.int32, (RD, FD * RD), 1)
        tmat = (rr == cc % RD).astype(jnp.float32)
        raw = jnp.dot(remb_rep, tmat, preferred_element_type=jnp.float32)
        mr = lax.broadcasted_iota(jnp.int32, (NR * FD, FD * RD), 0)
        mc = lax.broadcasted_iota(jnp.int32, (NR * FD, FD * RD), 1)
        e_all = raw * (mr % FD == mc // RD).astype(jnp.float32)
        m_scr[...] = jnp.zeros((GW, OUT), jnp.float32)
        m_scr[pl.ds(0, NR * FD), :] = lax.dot_general(
            e_all, w_ref[...], (((1,), (1,)), ((), ())),
            preferred_element_type=jnp.float32)

    o_ref[...] = jnp.dot(g_ref[...], m_scr[...],
                         preferred_element_type=jnp.float32) + b_ref[...]


def _tc_compute(g2, role_emb, w, b2):
    return pl.pallas_call(
        _tc_body,
        grid=(GRID,),
        in_specs=[
            pl.BlockSpec((NB_BLK, GW), lambda i: (i, 0)),
            pl.BlockSpec((NR, RD), lambda i: (0, 0)),
            pl.BlockSpec((OUT, FD * RD), lambda i: (0, 0)),
            pl.BlockSpec((1, OUT), lambda i: (0, 0)),
        ],
        out_specs=pl.BlockSpec((NB_BLK, OUT), lambda i: (i, 0)),
        out_shape=jax.ShapeDtypeStruct((B, OUT), jnp.float32),
        scratch_shapes=[pltpu.VMEM((GW, OUT), jnp.float32)],
    )(g2, role_emb, w, b2)


@jax.jit
def kernel(filler_list, role_list, filler_emb, role_emb, W, b):
    fpad, rpad = _pad_idx(filler_list, role_list)
    f_idx, r_idx = _sc_densify(fpad, rpad)
    g = _sc_bind(filler_emb, f_idx, r_idx)
    return _tc_compute(g.reshape(B, GW), role_emb, W, b.reshape(1, -1))


# speedup vs baseline: 1.1600x; 1.1550x over previous
import functools

import jax
import jax.numpy as jnp
from jax import lax
from jax.experimental import pallas as pl
from jax.experimental.pallas import tpu as pltpu
from jax.experimental.pallas import tpu_sc as plsc

B, S = 4096, 50
N = B * S
FD, RD, OUT = 32, 32, 128
NR = 50
KP = 52
GW = KP * FD

NC, NS = 2, 16
NW = NC * NS
BATCH_W = B // NW
NSUPER = 4
BATCH_SUP = BATCH_W // NSUPER
NCHUNK = 4
BATCH_CH = BATCH_SUP // NCHUNK
CH = BATCH_CH * S
GROWS_SUP = BATCH_SUP * KP
NZB = GROWS_SUP // 832
NSCAT = 5
SCAT = CH // NSCAT

NB_BLK = 128
GRID = B // NB_BLK
PAD_BLK = 512


def _sc_densify(fpad, rpad):
    mesh = plsc.VectorSubcoreMesh(core_axis_name="c", subcore_axis_name="s")

    @functools.partial(
        pl.kernel,
        out_type=(jax.ShapeDtypeStruct((N,), jnp.int32),
                  jax.ShapeDtypeStruct((N,), jnp.int32)),
        mesh=mesh,
        scratch_types=[
            pltpu.VMEM((BATCH_CH, 128), jnp.int32),
            pltpu.VMEM((BATCH_CH, 128), jnp.int32),
            pltpu.VMEM((CH,), jnp.int32),
            pltpu.VMEM((CH,), jnp.int32),
        ],
        compiler_params=pltpu.CompilerParams(use_tc_tiling_on_sc=True),
    )
    def k(fi_hbm, ri_hbm, fo_hbm, ro_hbm, fpad_v, rpad_v, fdense_v, rdense_v):
        cid = lax.axis_index("c")
        sid = lax.axis_index("s")
        wid = sid * NC + cid

        @pl.loop(0, BATCH_W // BATCH_CH)
        def _(cc):
            b0 = wid * BATCH_W + cc * BATCH_CH
            pltpu.sync_copy(fi_hbm.at[pl.ds(b0, BATCH_CH)], fpad_v)
            pltpu.sync_copy(ri_hbm.at[pl.ds(b0, BATCH_CH)], rpad_v)

            @pl.loop(0, BATCH_CH)
            def _(bi):
                for c in (0, 16, 32, 34):
                    fdense_v[pl.ds(bi * S + c, 16)] = fpad_v[bi, pl.ds(c, 16)]
                    rdense_v[pl.ds(bi * S + c, 16)] = rpad_v[bi, pl.ds(c, 16)]

            pltpu.sync_copy(fdense_v, fo_hbm.at[pl.ds(b0 * S, CH)])
            pltpu.sync_copy(rdense_v, ro_hbm.at[pl.ds(b0 * S, CH)])

    return k(fpad, rpad)

def _sc_bind(table, f_idx, r_idx):
    mesh = plsc.VectorSubcoreMesh(core_axis_name="c", subcore_axis_name="s")

    @functools.partial(
        pl.kernel,
        out_type=jax.ShapeDtypeStruct((B * KP, FD), jnp.float32),
        mesh=mesh,
        scratch_types=[
            pltpu.VMEM((CH,), jnp.int32),
            pltpu.VMEM((CH,), jnp.int32),
            pltpu.VMEM((NSCAT, SCAT), jnp.int32),
            pltpu.VMEM((CH, FD), jnp.float32),
            pltpu.VMEM((832, FD), jnp.float32),
            pltpu.VMEM_SHARED((NS, GROWS_SUP, FD), jnp.float32),
        ],
        compiler_params=pltpu.CompilerParams(use_tc_tiling_on_sc=False),
    )
    def k(tab_hbm, fi_hbm, ri_hbm, g_hbm,
          fidx_v, ridx_v, tgt_v, rows_v, zeros_v, acc_sh):
        cid = lax.axis_index("c")
        sid = lax.axis_index("s")
        wid = sid * NC + cid
        iota16 = lax.broadcasted_iota(jnp.int32, (16,), 0)
        z16 = jnp.zeros((16,), jnp.float32)

        @pl.loop(0, 832)
        def _(i):
            zeros_v[i, pl.ds(0, 16)] = z16
            zeros_v[i, pl.ds(16, 16)] = z16

        acc = acc_sh.at[sid]

        @pl.loop(0, NSUPER)
        def _(h):
            b_sup = wid * BATCH_W + h * BATCH_SUP

            @pl.loop(0, NZB)
            def _(zb):
                pltpu.sync_copy(zeros_v, acc.at[pl.ds(zb * 832, 832)])

            @pl.loop(0, NCHUNK)
            def _(cc):
                goff = (b_sup + cc * BATCH_CH) * S
                pltpu.sync_copy(fi_hbm.at[pl.ds(goff, CH)], fidx_v)
                pltpu.sync_copy(ri_hbm.at[pl.ds(goff, CH)], ridx_v)
                pltpu.sync_copy(tab_hbm.at[fidx_v], rows_v)

                @pl.loop(0, NSCAT)
                def _(j):
                    for t in range(SCAT // 16):
                        r0 = j * SCAT + t * 16
                        role16 = ridx_v[pl.ds(r0, 16)]
                        bloc = lax.shift_right_logical(
                            (r0 + iota16) * 1311, 16)
                        tgt = bloc * KP + cc * (BATCH_CH * KP) + role16
                        tgt_v[j, pl.ds(t * 16, 16)] = tgt

                @pl.loop(0, NSCAT)
                def _(j):
                    pltpu.sync_copy(rows_v.at[pl.ds(j * SCAT, SCAT)],
                                    acc.at[tgt_v.at[j]], add=True)

            pltpu.sync_copy(acc, g_hbm.at[pl.ds(b_sup * KP, GROWS_SUP)])

    return k(table, f_idx, r_idx)


TBLK = 4096
NTB = -(-1000000 // TBLK)


def _xpose_body(x_ref, o_ref):
    xt = x_ref[...].T
    z = xt.reshape(TBLK // 4, 4, FD)
    o_ref[...] = jnp.concatenate([z[:, j, :] for j in range(4)], axis=1)


def _transpose_table(emb_t):
    return pl.pallas_call(
        _xpose_body,
        grid=(NTB,),
        in_specs=[pl.BlockSpec((FD, TBLK), lambda i: (0, i))],
        out_specs=pl.BlockSpec((TBLK // 4, 128), lambda i: (i, 0)),
        out_shape=jax.ShapeDtypeStruct((250000, 128), jnp.float32),
    )(emb_t)


def _pad_body(f_ref, r_ref, fo_ref, ro_ref):
    z = jnp.zeros((PAD_BLK, 128 - S), jnp.int32)
    fo_ref[...] = jnp.concatenate([f_ref[...], z], axis=1)
    ro_ref[...] = jnp.concatenate([r_ref[...], z], axis=1)


def _pad_idx(filler_list, role_list):
    return pl.pallas_call(
        _pad_body,
        grid=(B // PAD_BLK,),
        in_specs=[
            pl.BlockSpec((PAD_BLK, S), lambda i: (i, 0)),
            pl.BlockSpec((PAD_BLK, S), lambda i: (i, 0)),
        ],
        out_specs=[
            pl.BlockSpec((PAD_BLK, 128), lambda i: (i, 0)),
            pl.BlockSpec((PAD_BLK, 128), lambda i: (i, 0)),
        ],
        out_shape=[jax.ShapeDtypeStruct((B, 128), jnp.int32),
                   jax.ShapeDtypeStruct((B, 128), jnp.int32)],
    )(filler_list, role_list)


def _tc_body(g_ref, remb_ref, w_ref, b_ref, o_ref, m_scr):
    @pl.when(pl.program_id(0) == 0)
    def _():
        remb_rep = jnp.broadcast_to(
            remb_ref[...][:, None, :], (NR, FD, RD)).reshape(NR * FD, RD)
        rr = lax.broadcasted_iota(jnp.int32, (RD, FD * RD), 0)
        cc = lax.broadcasted_iota(jnp.int32, (RD, FD * RD), 1)
        tmat = (rr == cc % RD).astype(jnp.float32)
        raw = jnp.dot(remb_rep, tmat, preferred_element_type=jnp.float32)
        mr = lax.broadcasted_iota(jnp.int32, (NR * FD, FD * RD), 0)
        mc = lax.broadcasted_iota(jnp.int32, (NR * FD, FD * RD), 1)
        e_all = raw * (mr % FD == mc // RD).astype(jnp.float32)
        m_scr[...] = jnp.zeros((GW, OUT), jnp.float32)
        m_scr[pl.ds(0, NR * FD), :] = lax.dot_general(
            e_all, w_ref[...], (((1,), (1,)), ((), ())),
            preferred_element_type=jnp.float32)

    o_ref[...] = jnp.dot(g_ref[...], m_scr[...],
                         preferred_element_type=jnp.float32) + b_ref[...]


def _tc_compute(g2, role_emb, w, b2):
    return pl.pallas_call(
        _tc_body,
        grid=(GRID,),
        in_specs=[
            pl.BlockSpec((NB_BLK, GW), lambda i: (i, 0)),
            pl.BlockSpec((NR, RD), lambda i: (0, 0)),
            pl.BlockSpec((OUT, FD * RD), lambda i: (0, 0)),
            pl.BlockSpec((1, OUT), lambda i: (0, 0)),
        ],
        out_specs=pl.BlockSpec((NB_BLK, OUT), lambda i: (i, 0)),
        out_shape=jax.ShapeDtypeStruct((B, OUT), jnp.float32),
        scratch_shapes=[pltpu.VMEM((GW, OUT), jnp.float32)],
    )(g2, role_emb, w, b2)


@jax.jit
def kernel(filler_list, role_list, filler_emb, role_emb, W, b):
    fpad, rpad = _pad_idx(filler_list, role_list)
    f_idx, r_idx = _sc_densify(fpad, rpad)
    table_rm = _transpose_table(filler_emb.T).reshape(1000000, FD)
    g = _sc_bind(table_rm, f_idx, r_idx)
    return _tc_compute(g.reshape(B, GW), role_emb, W, b.reshape(1, -1))
